# Initial kernel scaffold; baseline (speedup 1.0000x reference)
import jax
import jax.numpy as jnp
from jax.experimental import pallas as pl


def kernel(x, ln_w, ln_b, q_w, kv_w, proj_w, proj_b, attn1, attn2):
    raise NotImplementedError("write your pallas kernel here")


# speedup vs baseline: 75.3279x; 75.3279x over previous
import numpy as np
import jax
import jax.numpy as jnp
from jax.experimental import pallas as pl
from jax.experimental.pallas import tpu as pltpu

B, N, D = 64, 256, 768
H = 8
HD = D // H
FS = 16
SCALE = HD ** (-0.5)
KCNT1 = max(1, int(N / 2))
KCNT2 = max(1, int(N / 3))

INT32_MIN = np.int32(-(2 ** 31))
_BITVALS = [INT32_MIN if b == 31 else np.int32(1 << b) for b in range(31, -1, -1)]


def _pool_matrix() -> np.ndarray:
    P = np.zeros((N, N), np.float64)
    for k, p in ((3, 1), (5, 2), (7, 3)):
        A = np.zeros((FS, FS), np.float64)
        for i in range(FS):
            for j in range(max(0, i - p), min(FS, i + p + 1)):
                A[i, j] = 1.0 / k
        P += np.kron(A, A)
    return P.astype(np.float32)


_POOL_P = jnp.asarray(_pool_matrix())


def _kth_masks(key):
    ut1 = jnp.zeros((N, 1), jnp.int32)
    ut2 = jnp.zeros((N, 1), jnp.int32)
    for bit in _BITVALS:
        c1 = ut1 | bit
        c2 = ut2 | bit
        cnt1 = jnp.sum((key >= (c1 ^ INT32_MIN)).astype(jnp.float32),
                       axis=-1, keepdims=True)
        cnt2 = jnp.sum((key >= (c2 ^ INT32_MIN)).astype(jnp.float32),
                       axis=-1, keepdims=True)
        ut1 = jnp.where(cnt1 >= KCNT1, c1, ut1)
        ut2 = jnp.where(cnt2 >= KCNT2, c2, ut2)
    m1 = key >= (ut1 ^ INT32_MIN)
    m2 = key >= (ut2 ^ INT32_MIN)
    return m1, m2


def _body(x_ref, p_ref, lnw_ref, lnb_ref, qwt_ref, kvwt_ref, pwt_ref,
          pb_ref, a1_ref, a2_ref, out_ref):
    x = x_ref[0]
    y = jnp.dot(p_ref[...], x, preferred_element_type=jnp.float32, precision=jax.lax.Precision.HIGHEST)
    mu = jnp.mean(y, axis=-1, keepdims=True)
    var = jnp.mean((y - mu) * (y - mu), axis=-1, keepdims=True)
    yn = (y - mu) * jax.lax.rsqrt(var + 1e-5) * lnw_ref[...] + lnb_ref[...]
    kv = jnp.dot(yn, kvwt_ref[...], preferred_element_type=jnp.float32)
    q = jnp.dot(x, qwt_ref[...], preferred_element_type=jnp.float32)
    a1 = a1_ref[0, 0]
    a2 = a2_ref[0, 0]

    outs = []
    for h in range(H):
        qh = q[:, h * HD:(h + 1) * HD]
        kh = kv[:, h * HD:(h + 1) * HD]
        vh = kv[:, D + h * HD:D + (h + 1) * HD]
        attn = jax.lax.dot_general(
            qh, kh, (((1,), (1,)), ((), ())),
            preferred_element_type=jnp.float32) * SCALE
        ikey = jax.lax.bitcast_convert_type(attn, jnp.int32)
        key = jnp.where(ikey >= 0, ikey, INT32_MIN - ikey)
        m1, m2 = _kth_masks(key)
        rowmax = jnp.max(attn, axis=-1, keepdims=True)
        e = jnp.exp(attn - rowmax)
        e1 = jnp.where(m1, e, 0.0)
        e2 = jnp.where(m2, e, 0.0)
        s1 = jnp.sum(e1, axis=-1, keepdims=True)
        s2 = jnp.sum(e2, axis=-1, keepdims=True)
        w = e1 * (a1 / s1) + e2 * (a2 / s2)
        outs.append(jnp.dot(w, vh, preferred_element_type=jnp.float32))
    o = jnp.concatenate(outs, axis=-1)
    out_ref[0] = jnp.dot(o, pwt_ref[...],
                         preferred_element_type=jnp.float32) + pb_ref[...]


def kernel(x, ln_w, ln_b, q_w, kv_w, proj_w, proj_b, attn1, attn2):
    lnw = ln_w.reshape(1, D)
    lnb = ln_b.reshape(1, D)
    qwt = q_w.T
    kvwt = kv_w.T
    pwt = proj_w.T
    pb = proj_b.reshape(1, D)
    a1 = attn1.reshape(1, 1)
    a2 = attn2.reshape(1, 1)

    const = lambda *_: (0, 0)
    return pl.pallas_call(
        _body,
        grid=(B,),
        in_specs=[
            pl.BlockSpec((1, N, D), lambda b: (b, 0, 0)),
            pl.BlockSpec((N, N), const),
            pl.BlockSpec((1, D), const),
            pl.BlockSpec((1, D), const),
            pl.BlockSpec((D, D), const),
            pl.BlockSpec((D, 2 * D), const),
            pl.BlockSpec((D, D), const),
            pl.BlockSpec((1, D), const),
            pl.BlockSpec((1, 1), const),
            pl.BlockSpec((1, 1), const),
        ],
        out_specs=pl.BlockSpec((1, N, D), lambda b: (b, 0, 0)),
        out_shape=jax.ShapeDtypeStruct((B, N, D), jnp.float32),
        compiler_params=pltpu.CompilerParams(
            dimension_semantics=("parallel",),
        ),
    )(x, _POOL_P, lnw, lnb, qwt, kvwt, pwt, pb, a1, a2)


# speedup vs baseline: 121.3710x; 1.6112x over previous
import numpy as np
import jax
import jax.numpy as jnp
from jax.experimental import pallas as pl
from jax.experimental.pallas import tpu as pltpu

B, N, D = 64, 256, 768
H = 8
HD = D // H
FS = 16
SCALE = HD ** (-0.5)
KCNT1 = float(max(1, int(N / 2)))
KCNT2 = float(max(1, int(N / 3)))
BISECT_STEPS = 24


def _pool_matrix() -> np.ndarray:
    P = np.zeros((N, N), np.float64)
    for k, p in ((3, 1), (5, 2), (7, 3)):
        A = np.zeros((FS, FS), np.float64)
        for i in range(FS):
            for j in range(max(0, i - p), min(FS, i + p + 1)):
                A[i, j] = 1.0 / k
        P += np.kron(A, A)
    return P.astype(np.float32)


_POOL_P = _pool_matrix()


def _kth_threshold(at, lo0, hi0, kcnt):
    lo, hi = lo0, hi0
    for _ in range(BISECT_STEPS):
        mid = 0.5 * (lo + hi)
        cnt = jnp.sum((at >= mid).astype(jnp.float32), axis=0, keepdims=True)
        sel = cnt >= kcnt
        lo = jnp.where(sel, mid, lo)
        hi = jnp.where(sel, hi, mid)
    return lo


def _body(x_ref, p_ref, lnw_ref, lnb_ref, qwt_ref, kvwt_ref, pwt_ref,
          pb_ref, a1_ref, a2_ref, out_ref):
    x = x_ref[0]
    y = jnp.dot(p_ref[...], x, preferred_element_type=jnp.float32,
                precision=jax.lax.Precision.HIGHEST)
    mu = jnp.mean(y, axis=-1, keepdims=True)
    var = jnp.mean((y - mu) * (y - mu), axis=-1, keepdims=True)
    yn = (y - mu) * jax.lax.rsqrt(var + 1e-5) * lnw_ref[...] + lnb_ref[...]
    kv = jnp.dot(yn, kvwt_ref[...], preferred_element_type=jnp.float32)
    q = jnp.dot(x, qwt_ref[...], preferred_element_type=jnp.float32)
    a1 = a1_ref[0, 0]
    a2 = a2_ref[0, 0]

    outs = []
    for h in range(H):
        qh = q[:, h * HD:(h + 1) * HD]
        kh = kv[:, h * HD:(h + 1) * HD]
        vh = kv[:, D + h * HD:D + (h + 1) * HD]
        at = jax.lax.dot_general(
            kh, qh, (((1,), (1,)), ((), ())),
            preferred_element_type=jnp.float32) * SCALE
        hi0 = jnp.max(at, axis=0, keepdims=True)
        lo0 = jnp.min(at, axis=0, keepdims=True)
        t1 = _kth_threshold(at, lo0, hi0, KCNT1)
        t2 = _kth_threshold(at, lo0, hi0, KCNT2)
        e = jnp.exp(at - hi0)
        e1 = jnp.where(at >= t1, e, 0.0)
        e2 = jnp.where(at >= t2, e, 0.0)
        s1 = jnp.sum(e1, axis=0, keepdims=True)
        s2 = jnp.sum(e2, axis=0, keepdims=True)
        w = e1 * (a1 / s1) + e2 * (a2 / s2)
        outs.append(jax.lax.dot_general(
            w, vh, (((0,), (0,)), ((), ())),
            preferred_element_type=jnp.float32))
    o = jnp.concatenate(outs, axis=-1)
    out_ref[0] = jnp.dot(o, pwt_ref[...],
                         preferred_element_type=jnp.float32) + pb_ref[...]


def kernel(x, ln_w, ln_b, q_w, kv_w, proj_w, proj_b, attn1, attn2):
    lnw = ln_w.reshape(1, D)
    lnb = ln_b.reshape(1, D)
    qwt = q_w.T
    kvwt = kv_w.T
    pwt = proj_w.T
    pb = proj_b.reshape(1, D)
    a1 = attn1.reshape(1, 1)
    a2 = attn2.reshape(1, 1)

    const = lambda *_: (0, 0)
    return pl.pallas_call(
        _body,
        grid=(B,),
        in_specs=[
            pl.BlockSpec((1, N, D), lambda b: (b, 0, 0)),
            pl.BlockSpec((N, N), const),
            pl.BlockSpec((1, D), const),
            pl.BlockSpec((1, D), const),
            pl.BlockSpec((D, D), const),
            pl.BlockSpec((D, 2 * D), const),
            pl.BlockSpec((D, D), const),
            pl.BlockSpec((1, D), const),
            pl.BlockSpec((1, 1), const),
            pl.BlockSpec((1, 1), const),
        ],
        out_specs=pl.BlockSpec((1, N, D), lambda b: (b, 0, 0)),
        out_shape=jax.ShapeDtypeStruct((B, N, D), jnp.float32),
        compiler_params=pltpu.CompilerParams(
            dimension_semantics=("parallel",),
        ),
    )(x, _POOL_P, lnw, lnb, qwt, kvwt, pwt, pb, a1, a2)


# speedup vs baseline: 136.0701x; 1.1211x over previous
import numpy as np
import jax
import jax.numpy as jnp
from jax.experimental import pallas as pl
from jax.experimental.pallas import tpu as pltpu

B, N, D = 64, 256, 768
H = 8
HD = D // H
FS = 16
SCALE = HD ** (-0.5)
KCNT1 = float(max(1, int(N / 2)))
KCNT2 = float(max(1, int(N / 3)))
BISECT_STEPS = 22


def _pool_matrix() -> np.ndarray:
    P = np.zeros((N, N), np.float64)
    for k, p in ((3, 1), (5, 2), (7, 3)):
        A = np.zeros((FS, FS), np.float64)
        for i in range(FS):
            for j in range(max(0, i - p), min(FS, i + p + 1)):
                A[i, j] = 1.0 / k
        P += np.kron(A, A)
    return P.astype(np.float32)


_POOL_P = _pool_matrix()


def _kth_thresholds(at, lo0, hi0):
    lo1, hi1 = lo0, hi0
    lo2, hi2 = lo0, hi0
    for _ in range(BISECT_STEPS):
        mid1 = 0.5 * (lo1 + hi1)
        mid2 = 0.5 * (lo2 + hi2)
        cnt1 = jnp.sum((at >= mid1).astype(jnp.float32), axis=0, keepdims=True)
        cnt2 = jnp.sum((at >= mid2).astype(jnp.float32), axis=0, keepdims=True)
        sel1 = cnt1 >= KCNT1
        sel2 = cnt2 >= KCNT2
        lo1 = jnp.where(sel1, mid1, lo1)
        hi1 = jnp.where(sel1, hi1, mid1)
        lo2 = jnp.where(sel2, mid2, lo2)
        hi2 = jnp.where(sel2, hi2, mid2)
    return lo1, lo2


def _body(x_ref, p_ref, lnw_ref, lnb_ref, qwt_ref, kvwt_ref, pwt_ref,
          pb_ref, a1_ref, a2_ref, out_ref):
    x = x_ref[0]
    y = jnp.dot(p_ref[...], x, preferred_element_type=jnp.float32,
                precision=jax.lax.Precision.HIGHEST)
    mu = jnp.mean(y, axis=-1, keepdims=True)
    var = jnp.mean((y - mu) * (y - mu), axis=-1, keepdims=True)
    yn = (y - mu) * jax.lax.rsqrt(var + 1e-5) * lnw_ref[...] + lnb_ref[...]
    kv = jnp.dot(yn, kvwt_ref[...], preferred_element_type=jnp.float32)
    q = jnp.dot(x, qwt_ref[...], preferred_element_type=jnp.float32)
    a1 = a1_ref[0, 0]
    a2 = a2_ref[0, 0]

    outs = []
    for h in range(H):
        qh = q[:, h * HD:(h + 1) * HD]
        kh = kv[:, h * HD:(h + 1) * HD]
        vh = kv[:, D + h * HD:D + (h + 1) * HD]
        at = jax.lax.dot_general(
            kh, qh, (((1,), (1,)), ((), ())),
            preferred_element_type=jnp.float32) * SCALE
        hi0 = jnp.max(at, axis=0, keepdims=True)
        lo0 = jnp.min(at, axis=0, keepdims=True)
        t1, t2 = _kth_thresholds(at, lo0, hi0)
        e = jnp.exp(at - hi0)
        e1 = jnp.where(at >= t1, e, 0.0)
        e2 = jnp.where(at >= t2, e, 0.0)
        s1 = jnp.sum(e1, axis=0, keepdims=True)
        s2 = jnp.sum(e2, axis=0, keepdims=True)
        w = e1 * (a1 / s1) + e2 * (a2 / s2)
        outs.append(jax.lax.dot_general(
            w, vh, (((0,), (0,)), ((), ())),
            preferred_element_type=jnp.float32))
    o = jnp.concatenate(outs, axis=-1)
    out_ref[0] = jnp.dot(o, pwt_ref[...],
                         preferred_element_type=jnp.float32) + pb_ref[...]


def kernel(x, ln_w, ln_b, q_w, kv_w, proj_w, proj_b, attn1, attn2):
    lnw = ln_w.reshape(1, D)
    lnb = ln_b.reshape(1, D)
    qwt = q_w.T
    kvwt = kv_w.T
    pwt = proj_w.T
    pb = proj_b.reshape(1, D)
    a1 = attn1.reshape(1, 1)
    a2 = attn2.reshape(1, 1)

    const = lambda *_: (0, 0)
    return pl.pallas_call(
        _body,
        grid=(B,),
        in_specs=[
            pl.BlockSpec((1, N, D), lambda b: (b, 0, 0)),
            pl.BlockSpec((N, N), const),
            pl.BlockSpec((1, D), const),
            pl.BlockSpec((1, D), const),
            pl.BlockSpec((D, D), const),
            pl.BlockSpec((D, 2 * D), const),
            pl.BlockSpec((D, D), const),
            pl.BlockSpec((1, D), const),
            pl.BlockSpec((1, 1), const),
            pl.BlockSpec((1, 1), const),
        ],
        out_specs=pl.BlockSpec((1, N, D), lambda b: (b, 0, 0)),
        out_shape=jax.ShapeDtypeStruct((B, N, D), jnp.float32),
        compiler_params=pltpu.CompilerParams(
            dimension_semantics=("parallel",),
        ),
    )(x, _POOL_P, lnw, lnb, qwt, kvwt, pwt, pb, a1, a2)


# speedup vs baseline: 140.2349x; 1.0306x over previous
import numpy as np
import jax
import jax.numpy as jnp
from jax.experimental import pallas as pl
from jax.experimental.pallas import tpu as pltpu

B, N, D = 64, 256, 768
H = 8
HD = D // H
FS = 16
SCALE = HD ** (-0.5)
KCNT1 = float(max(1, int(N / 2)))
KCNT2 = float(max(1, int(N / 3)))
BISECT_STEPS = 22


def _pool_matrix() -> np.ndarray:
    P = np.zeros((N, N), np.float64)
    for k, p in ((3, 1), (5, 2), (7, 3)):
        A = np.zeros((FS, FS), np.float64)
        for i in range(FS):
            for j in range(max(0, i - p), min(FS, i + p + 1)):
                A[i, j] = 1.0 / k
        P += np.kron(A, A)
    return P.astype(np.float32)


_POOL_P = _pool_matrix()


def _kth_thresholds(at, lo0, hi0):
    lo1, hi1 = lo0, hi0
    lo2, hi2 = lo0, hi0
    for _ in range(BISECT_STEPS):
        mid1 = 0.5 * (lo1 + hi1)
        mid2 = 0.5 * (lo2 + hi2)
        ind = jnp.where(at >= mid2, 513.0,
                        jnp.where(at >= mid1, 1.0, 0.0))
        cnt = jnp.sum(ind, axis=0, keepdims=True)
        cnt2 = jnp.floor(cnt * (1.0 / 512.0))
        cnt1 = cnt - 512.0 * cnt2
        sel1 = cnt1 >= KCNT1
        sel2 = cnt2 >= KCNT2
        lo1 = jnp.where(sel1, mid1, lo1)
        hi1 = jnp.where(sel1, hi1, mid1)
        lo2 = jnp.where(sel2, mid2, lo2)
        hi2 = jnp.where(sel2, hi2, mid2)
        lo2 = jnp.maximum(lo2, lo1)
        hi1 = jnp.minimum(hi1, hi2)
    return lo1, lo2


def _body(x_ref, p_ref, lnw_ref, lnb_ref, qwt_ref, kvwt_ref, pwt_ref,
          pb_ref, a1_ref, a2_ref, out_ref):
    x = x_ref[0]
    y = jnp.dot(p_ref[...], x, preferred_element_type=jnp.float32,
                precision=jax.lax.Precision.HIGHEST)
    mu = jnp.mean(y, axis=-1, keepdims=True)
    var = jnp.mean((y - mu) * (y - mu), axis=-1, keepdims=True)
    yn = (y - mu) * jax.lax.rsqrt(var + 1e-5) * lnw_ref[...] + lnb_ref[...]
    kv = jnp.dot(yn, kvwt_ref[...], preferred_element_type=jnp.float32)
    q = jnp.dot(x, qwt_ref[...], preferred_element_type=jnp.float32)
    a1 = a1_ref[0, 0]
    a2 = a2_ref[0, 0]

    outs = []
    for h in range(H):
        qh = q[:, h * HD:(h + 1) * HD]
        kh = kv[:, h * HD:(h + 1) * HD]
        vh = kv[:, D + h * HD:D + (h + 1) * HD]
        at = jax.lax.dot_general(
            kh, qh, (((1,), (1,)), ((), ())),
            preferred_element_type=jnp.float32) * SCALE
        hi0 = jnp.max(at, axis=0, keepdims=True)
        lo0 = jnp.min(at, axis=0, keepdims=True)
        t1, t2 = _kth_thresholds(at, lo0, hi0)
        e = jnp.exp(at - hi0)
        e1 = jnp.where(at >= t1, e, 0.0)
        e2 = jnp.where(at >= t2, e, 0.0)
        s1 = jnp.sum(e1, axis=0, keepdims=True)
        s2 = jnp.sum(e2, axis=0, keepdims=True)
        w = e1 * (a1 / s1) + e2 * (a2 / s2)
        outs.append(jax.lax.dot_general(
            w, vh, (((0,), (0,)), ((), ())),
            preferred_element_type=jnp.float32))
    o = jnp.concatenate(outs, axis=-1)
    out_ref[0] = jnp.dot(o, pwt_ref[...],
                         preferred_element_type=jnp.float32) + pb_ref[...]


def kernel(x, ln_w, ln_b, q_w, kv_w, proj_w, proj_b, attn1, attn2):
    lnw = ln_w.reshape(1, D)
    lnb = ln_b.reshape(1, D)
    qwt = q_w.T
    kvwt = kv_w.T
    pwt = proj_w.T
    pb = proj_b.reshape(1, D)
    a1 = attn1.reshape(1, 1)
    a2 = attn2.reshape(1, 1)

    const = lambda *_: (0, 0)
    return pl.pallas_call(
        _body,
        grid=(B,),
        in_specs=[
            pl.BlockSpec((1, N, D), lambda b: (b, 0, 0)),
            pl.BlockSpec((N, N), const),
            pl.BlockSpec((1, D), const),
            pl.BlockSpec((1, D), const),
            pl.BlockSpec((D, D), const),
            pl.BlockSpec((D, 2 * D), const),
            pl.BlockSpec((D, D), const),
            pl.BlockSpec((1, D), const),
            pl.BlockSpec((1, 1), const),
            pl.BlockSpec((1, 1), const),
        ],
        out_specs=pl.BlockSpec((1, N, D), lambda b: (b, 0, 0)),
        out_shape=jax.ShapeDtypeStruct((B, N, D), jnp.float32),
        compiler_params=pltpu.CompilerParams(
            dimension_semantics=("parallel",),
        ),
    )(x, _POOL_P, lnw, lnb, qwt, kvwt, pwt, pb, a1, a2)


# speedup vs baseline: 170.3349x; 1.2146x over previous
import numpy as np
import jax
import jax.numpy as jnp
from jax.experimental import pallas as pl
from jax.experimental.pallas import tpu as pltpu

B, N, D = 64, 256, 768
H = 8
HD = D // H
FS = 16
SCALE = HD ** (-0.5)
KCNT1 = float(max(1, int(N / 2)))
KCNT2 = float(max(1, int(N / 3)))
BISECT_STEPS = 16


def _pool_matrix() -> np.ndarray:
    P = np.zeros((N, N), np.float64)
    for k, p in ((3, 1), (5, 2), (7, 3)):
        A = np.zeros((FS, FS), np.float64)
        for i in range(FS):
            for j in range(max(0, i - p), min(FS, i + p + 1)):
                A[i, j] = 1.0 / k
        P += np.kron(A, A)
    return P.astype(np.float32)


_POOL_P = _pool_matrix()


def _kth_thresholds(at, lo0, hi0):
    lo1, hi1 = lo0, hi0
    lo2, hi2 = lo0, hi0
    for _ in range(BISECT_STEPS):
        mid1 = 0.5 * (lo1 + hi1)
        mid2 = 0.5 * (lo2 + hi2)
        ind = jnp.where(at >= mid2, 513.0,
                        jnp.where(at >= mid1, 1.0, 0.0))
        cnt = jnp.sum(ind, axis=0, keepdims=True)
        cnt2 = jnp.floor(cnt * (1.0 / 512.0))
        cnt1 = cnt - 512.0 * cnt2
        sel1 = cnt1 >= KCNT1
        sel2 = cnt2 >= KCNT2
        lo1 = jnp.where(sel1, mid1, lo1)
        hi1 = jnp.where(sel1, hi1, mid1)
        lo2 = jnp.where(sel2, mid2, lo2)
        hi2 = jnp.where(sel2, hi2, mid2)
        lo2 = jnp.maximum(lo2, lo1)
        hi1 = jnp.minimum(hi1, hi2)
    return lo1, lo2


def _body(x_ref, p_ref, lnw_ref, lnb_ref, qwt_ref, kvwt_ref, pwt_ref,
          pb_ref, a1_ref, a2_ref, out_ref):
    x = x_ref[0]
    y = jnp.dot(p_ref[...], x, preferred_element_type=jnp.float32,
                precision=jax.lax.Precision.HIGHEST)
    mu = jnp.mean(y, axis=-1, keepdims=True)
    var = jnp.mean((y - mu) * (y - mu), axis=-1, keepdims=True)
    yn = (y - mu) * jax.lax.rsqrt(var + 1e-5) * lnw_ref[...] + lnb_ref[...]
    kv = jnp.dot(yn, kvwt_ref[...], preferred_element_type=jnp.float32)
    q = jnp.dot(x, qwt_ref[...], preferred_element_type=jnp.float32)
    a1 = a1_ref[0, 0]
    a2 = a2_ref[0, 0]

    outs = []
    for h in range(H):
        qh = q[:, h * HD:(h + 1) * HD]
        kh = kv[:, h * HD:(h + 1) * HD]
        vh = kv[:, D + h * HD:D + (h + 1) * HD]
        at = jax.lax.dot_general(
            kh, qh, (((1,), (1,)), ((), ())),
            preferred_element_type=jnp.float32) * SCALE
        hi0 = jnp.max(at, axis=0, keepdims=True)
        lo0 = jnp.min(at, axis=0, keepdims=True)
        t1, t2 = _kth_thresholds(at, lo0, hi0)
        e = jnp.exp(at - hi0)
        e1 = jnp.where(at >= t1, e, 0.0)
        e2 = jnp.where(at >= t2, e, 0.0)
        s1 = jnp.sum(e1, axis=0, keepdims=True)
        s2 = jnp.sum(e2, axis=0, keepdims=True)
        w = e1 * (a1 / s1) + e2 * (a2 / s2)
        outs.append(jax.lax.dot_general(
            w, vh, (((0,), (0,)), ((), ())),
            preferred_element_type=jnp.float32))
    o = jnp.concatenate(outs, axis=-1)
    out_ref[0] = jnp.dot(o, pwt_ref[...],
                         preferred_element_type=jnp.float32) + pb_ref[...]


def kernel(x, ln_w, ln_b, q_w, kv_w, proj_w, proj_b, attn1, attn2):
    lnw = ln_w.reshape(1, D)
    lnb = ln_b.reshape(1, D)
    qwt = q_w.T
    kvwt = kv_w.T
    pwt = proj_w.T
    pb = proj_b.reshape(1, D)
    a1 = attn1.reshape(1, 1)
    a2 = attn2.reshape(1, 1)

    const = lambda *_: (0, 0)
    return pl.pallas_call(
        _body,
        grid=(B,),
        in_specs=[
            pl.BlockSpec((1, N, D), lambda b: (b, 0, 0)),
            pl.BlockSpec((N, N), const),
            pl.BlockSpec((1, D), const),
            pl.BlockSpec((1, D), const),
            pl.BlockSpec((D, D), const),
            pl.BlockSpec((D, 2 * D), const),
            pl.BlockSpec((D, D), const),
            pl.BlockSpec((1, D), const),
            pl.BlockSpec((1, 1), const),
            pl.BlockSpec((1, 1), const),
        ],
        out_specs=pl.BlockSpec((1, N, D), lambda b: (b, 0, 0)),
        out_shape=jax.ShapeDtypeStruct((B, N, D), jnp.float32),
        compiler_params=pltpu.CompilerParams(
            dimension_semantics=("parallel",),
        ),
    )(x, _POOL_P, lnw, lnb, qwt, kvwt, pwt, pb, a1, a2)


# speedup vs baseline: 182.9680x; 1.0742x over previous
import numpy as np
import jax
import jax.numpy as jnp
from jax.experimental import pallas as pl
from jax.experimental.pallas import tpu as pltpu

B, N, D = 64, 256, 768
H = 8
HD = D // H
FS = 16
SCALE = HD ** (-0.5)
KCNT1 = float(max(1, int(N / 2)))
KCNT2 = float(max(1, int(N / 3)))
BISECT_STEPS = 14


def _pool_matrix() -> np.ndarray:
    P = np.zeros((N, N), np.float64)
    for k, p in ((3, 1), (5, 2), (7, 3)):
        A = np.zeros((FS, FS), np.float64)
        for i in range(FS):
            for j in range(max(0, i - p), min(FS, i + p + 1)):
                A[i, j] = 1.0 / k
        P += np.kron(A, A)
    return P.astype(np.float32)


_POOL_P = _pool_matrix()


def _kth_thresholds(at, lo0, hi0):
    lo1, hi1 = lo0, hi0
    lo2, hi2 = lo0, hi0
    for _ in range(BISECT_STEPS):
        mid1 = 0.5 * (lo1 + hi1)
        mid2 = 0.5 * (lo2 + hi2)
        ind = jnp.where(at >= mid2, 513.0,
                        jnp.where(at >= mid1, 1.0, 0.0))
        cnt = jnp.sum(ind, axis=0, keepdims=True)
        cnt2 = jnp.floor(cnt * (1.0 / 512.0))
        cnt1 = cnt - 512.0 * cnt2
        sel1 = cnt1 >= KCNT1
        sel2 = cnt2 >= KCNT2
        lo1 = jnp.where(sel1, mid1, lo1)
        hi1 = jnp.where(sel1, hi1, mid1)
        lo2 = jnp.where(sel2, mid2, lo2)
        hi2 = jnp.where(sel2, hi2, mid2)
        lo2 = jnp.maximum(lo2, lo1)
        hi1 = jnp.minimum(hi1, hi2)
    return lo1, lo2


def _body(x_ref, p_ref, lnw_ref, lnb_ref, qwt_ref, kvwt_ref, pwt_ref,
          pb_ref, a1_ref, a2_ref, out_ref):
    x = x_ref[0]
    y = jnp.dot(p_ref[...], x, preferred_element_type=jnp.float32,
                precision=jax.lax.Precision.HIGHEST)
    mu = jnp.mean(y, axis=-1, keepdims=True)
    var = jnp.mean((y - mu) * (y - mu), axis=-1, keepdims=True)
    yn = (y - mu) * jax.lax.rsqrt(var + 1e-5) * lnw_ref[...] + lnb_ref[...]
    kv = jnp.dot(yn, kvwt_ref[...], preferred_element_type=jnp.float32)
    q = jnp.dot(x, qwt_ref[...], preferred_element_type=jnp.float32)
    a1 = a1_ref[0, 0]
    a2 = a2_ref[0, 0]

    outs = []
    for h in range(H):
        qh = q[:, h * HD:(h + 1) * HD]
        kh = kv[:, h * HD:(h + 1) * HD]
        vh = kv[:, D + h * HD:D + (h + 1) * HD]
        at = jax.lax.dot_general(
            kh, qh, (((1,), (1,)), ((), ())),
            preferred_element_type=jnp.float32) * SCALE
        hi0 = jnp.max(at, axis=0, keepdims=True)
        lo0 = jnp.min(at, axis=0, keepdims=True)
        t1, t2 = _kth_thresholds(at, lo0, hi0)
        e = jnp.exp(at - hi0)
        e1 = jnp.where(at >= t1, e, 0.0)
        e2 = jnp.where(at >= t2, e, 0.0)
        s1 = jnp.sum(e1, axis=0, keepdims=True)
        s2 = jnp.sum(e2, axis=0, keepdims=True)
        w = e1 * (a1 / s1) + e2 * (a2 / s2)
        outs.append(jax.lax.dot_general(
            w, vh, (((0,), (0,)), ((), ())),
            preferred_element_type=jnp.float32))
    o = jnp.concatenate(outs, axis=-1)
    out_ref[0] = jnp.dot(o, pwt_ref[...],
                         preferred_element_type=jnp.float32) + pb_ref[...]


def kernel(x, ln_w, ln_b, q_w, kv_w, proj_w, proj_b, attn1, attn2):
    lnw = ln_w.reshape(1, D)
    lnb = ln_b.reshape(1, D)
    qwt = q_w.T
    kvwt = kv_w.T
    pwt = proj_w.T
    pb = proj_b.reshape(1, D)
    a1 = attn1.reshape(1, 1)
    a2 = attn2.reshape(1, 1)

    const = lambda *_: (0, 0)
    return pl.pallas_call(
        _body,
        grid=(B,),
        in_specs=[
            pl.BlockSpec((1, N, D), lambda b: (b, 0, 0)),
            pl.BlockSpec((N, N), const),
            pl.BlockSpec((1, D), const),
            pl.BlockSpec((1, D), const),
            pl.BlockSpec((D, D), const),
            pl.BlockSpec((D, 2 * D), const),
            pl.BlockSpec((D, D), const),
            pl.BlockSpec((1, D), const),
            pl.BlockSpec((1, 1), const),
            pl.BlockSpec((1, 1), const),
        ],
        out_specs=pl.BlockSpec((1, N, D), lambda b: (b, 0, 0)),
        out_shape=jax.ShapeDtypeStruct((B, N, D), jnp.float32),
        compiler_params=pltpu.CompilerParams(
            dimension_semantics=("parallel",),
        ),
    )(x, _POOL_P, lnw, lnb, qwt, kvwt, pwt, pb, a1, a2)
